# baseline (speedup 1.0000x reference)
import functools

import jax
import jax.numpy as jnp
from jax import lax
from jax.experimental import pallas as pl
from jax.experimental.pallas import tpu as pltpu
from jax.experimental.pallas import tpu_sc as plsc

B, N, K, D = 4, 10000, 32, 128
OUT, HID = 128, 256
BN = B * N
NW = 32
C = 128
CH_PER_W = 10
CHUNKS = NW * CH_PER_W
BN_PAD = CHUNKS * C
R = 1000
NC = N // R


def _sc_neighbor_sum(feat_flat, idx_chunks, W):
    mesh = plsc.VectorSubcoreMesh(core_axis_name="c", subcore_axis_name="s")

    @functools.partial(
        pl.kernel,
        out_type=jax.ShapeDtypeStruct((BN_PAD, W), jnp.float32),
        mesh=mesh,
        scratch_types=[
            pltpu.VMEM((K, C), jnp.int32),
            pltpu.VMEM((C, W), jnp.float32),
            pltpu.SemaphoreType.DMA,
        ],
    )
    def sc_kernel(feat_hbm, idx_hbm, out_hbm, idx_v, acc_v, sem):
        cid = lax.axis_index("c")
        sid = lax.axis_index("s")
        wid = sid * 2 + cid

        def chunk_body(j, carry):
            chunk = wid * CH_PER_W + j
            pltpu.sync_copy(idx_hbm.at[chunk], idx_v)
            pltpu.async_copy(feat_hbm.at[idx_v.at[0]], acc_v, sem).wait()

            def fire(k, c2):
                pltpu.async_copy(feat_hbm.at[idx_v.at[k]], acc_v, sem, add=True)
                return c2

            lax.fori_loop(1, K, fire, 0)

            def drain(k, c2):
                pltpu.make_async_copy(feat_hbm.at[pl.ds(0, C)], acc_v, sem).wait()
                return c2

            lax.fori_loop(1, K, drain, 0)

            pltpu.sync_copy(acc_v, out_hbm.at[pl.ds(chunk * C, C)])
            return carry

        lax.fori_loop(0, CH_PER_W, chunk_body, 0)

    return sc_kernel(feat_flat, idx_chunks)


def _tc_dense(feat, nsum, actions, w_top, w_bot, f1w, f1b, f2w, f2b, f3w, f3b):
    def body(actions_ref, feat_ref, nsum_ref, wt_ref, wb_ref,
             f1w_ref, f1b_ref, f2w_ref, f2b_ref, f3w_ref, f3b_ref,
             out_ref, xbuf):
        b = pl.program_id(0)
        c = pl.program_id(1)

        @pl.when(jnp.logical_and(b == 0, c == 0))
        def _init():
            xbuf[...] = jnp.zeros_like(xbuf)

        f = feat_ref[0]
        s = nsum_ref[...]
        e = jnp.dot(f, wt_ref[...], preferred_element_type=jnp.float32)
        e = e + jnp.dot(s, wb_ref[...], preferred_element_type=jnp.float32)
        e = jnp.maximum(e, 0.0)

        part_sum = jnp.sum(e, axis=0, keepdims=True)
        act = actions_ref[b]
        rows = lax.broadcasted_iota(jnp.int32, (R, OUT), 0) + c * R
        mask = (rows == act).astype(jnp.float32)
        part_act = jnp.sum(e * mask, axis=0, keepdims=True)
        upd = jnp.concatenate([part_sum, part_act], axis=1)
        xbuf[pl.ds(b, 1), :] = xbuf[pl.ds(b, 1), :] + upd

        @pl.when(jnp.logical_and(b == B - 1, c == NC - 1))
        def _head():
            scale = jnp.concatenate(
                [jnp.full((1, OUT), 1.0 / N, jnp.float32),
                 jnp.ones((1, OUT), jnp.float32)], axis=1)
            x = xbuf[...] * scale
            h = jnp.dot(x, f1w_ref[...], preferred_element_type=jnp.float32)
            h = jnp.maximum(h + f1b_ref[...], 0.0)
            h = jnp.dot(h, f2w_ref[...], preferred_element_type=jnp.float32)
            h = jnp.maximum(h + f2b_ref[...], 0.0)
            o = jnp.dot(h, f3w_ref[...], preferred_element_type=jnp.float32)
            out_ref[...] = o + f3b_ref[...]

    return pl.pallas_call(
        body,
        grid=(B, NC),
        in_specs=[
            pl.BlockSpec(memory_space=pltpu.SMEM),
            pl.BlockSpec((1, R, D), lambda b, c: (b, c, 0)),
            pl.BlockSpec((R, D), lambda b, c: (b * NC + c, 0)),
            pl.BlockSpec((D, OUT), lambda b, c: (0, 0)),
            pl.BlockSpec((D, OUT), lambda b, c: (0, 0)),
            pl.BlockSpec((2 * OUT, HID), lambda b, c: (0, 0)),
            pl.BlockSpec((1, HID), lambda b, c: (0, 0)),
            pl.BlockSpec((HID, HID), lambda b, c: (0, 0)),
            pl.BlockSpec((1, HID), lambda b, c: (0, 0)),
            pl.BlockSpec((HID, OUT), lambda b, c: (0, 0)),
            pl.BlockSpec((1, OUT), lambda b, c: (0, 0)),
        ],
        out_specs=pl.BlockSpec((8, OUT), lambda b, c: (0, 0)),
        out_shape=jax.ShapeDtypeStruct((8, OUT), jnp.float32),
        scratch_shapes=[pltpu.VMEM((8, 2 * OUT), jnp.float32)],
    )(actions, feat, nsum, w_top, w_bot, f1w, f1b, f2w, f2b, f3w, f3b)


def kernel(actions, features, adj_lists, nodes, W_sage, fc1_w, fc1_b, fc2_w, fc2_b, fc3_w, fc3_b):
    del nodes
    feat_flat = features.reshape(BN, D)

    adj = adj_lists.astype(jnp.int32) + (jnp.arange(B, dtype=jnp.int32) * N)[:, None, None]
    adj_flat = adj.reshape(BN, K)
    adj_pad = jnp.pad(adj_flat, ((0, BN_PAD - BN), (0, 0)))
    idx_chunks = adj_pad.reshape(CHUNKS, C, K).transpose(0, 2, 1)

    nsum = _sc_neighbor_sum(feat_flat, idx_chunks, D)

    w_top = W_sage[:D]
    w_bot = W_sage[D:] * (1.0 / K)
    f3w = jnp.pad(fc3_w, ((0, 0), (0, OUT - 1)))
    f3b = jnp.pad(fc3_b, (0, OUT - 1)).reshape(1, OUT)

    out8 = _tc_dense(features, nsum, actions.astype(jnp.int32),
                     w_top, w_bot,
                     fc1_w, fc1_b.reshape(1, HID),
                     fc2_w, fc2_b.reshape(1, HID),
                     f3w, f3b)
    return out8[:B, :1]


# speedup vs baseline: 6.3839x; 6.3839x over previous
import functools

import jax
import jax.numpy as jnp
from jax import lax
from jax.experimental import pallas as pl
from jax.experimental.pallas import tpu as pltpu
from jax.experimental.pallas import tpu_sc as plsc

B, N, K, D = 4, 10000, 32, 128
OUT, HID = 128, 256
NSUB = 16
C = 128
N_PAD = 10240
CH_PER_G = N_PAD // C
CH_PER_W = CH_PER_G // NSUB
G_PER_CORE = B // 2
STG = 640
R = 1000
NC = N // R


def _sc_neighbor_sum(feat, idx_chunks):
    mesh = plsc.VectorSubcoreMesh(core_axis_name="c", subcore_axis_name="s")

    @functools.partial(
        pl.kernel,
        out_type=jax.ShapeDtypeStruct((B, N_PAD, D), jnp.float32),
        mesh=mesh,
        scratch_types=[
            pltpu.VMEM_SHARED((N, D), jnp.float32),
            pltpu.VMEM((K, C), jnp.int32),
            pltpu.VMEM((C, D), jnp.float32),
            pltpu.SemaphoreType.DMA,
        ],
    )
    def sc_kernel(feat_hbm, idx_hbm, out_hbm, table, idx_v, acc_v, sem):
        cid = lax.axis_index("c")
        sid = lax.axis_index("s")

        for g in range(G_PER_CORE):
            b = cid * G_PER_CORE + g

            @pl.when(sid < NSUB - 1)
            def _stage_full():
                pltpu.sync_copy(feat_hbm.at[b, pl.ds(sid * STG, STG)],
                                table.at[pl.ds(sid * STG, STG)])

            @pl.when(sid == NSUB - 1)
            def _stage_tail():
                pltpu.sync_copy(feat_hbm.at[b, pl.ds((NSUB - 1) * STG, N - (NSUB - 1) * STG)],
                                table.at[pl.ds((NSUB - 1) * STG, N - (NSUB - 1) * STG)])

            plsc.subcore_barrier()

            def chunk_body(j, carry):
                chunk = sid * CH_PER_W + j
                pltpu.sync_copy(idx_hbm.at[b, chunk], idx_v)
                pltpu.async_copy(table.at[idx_v.at[0]], acc_v, sem).wait()

                def fire(k, c2):
                    pltpu.async_copy(table.at[idx_v.at[k]], acc_v, sem, add=True)
                    return c2

                lax.fori_loop(1, K, fire, 0)

                def drain(k, c2):
                    pltpu.make_async_copy(feat_hbm.at[0, pl.ds(0, C)], acc_v, sem).wait()
                    return c2

                lax.fori_loop(1, K, drain, 0)

                pltpu.sync_copy(acc_v, out_hbm.at[b, pl.ds(chunk * C, C)])
                return carry

            lax.fori_loop(0, CH_PER_W, chunk_body, 0)

            plsc.subcore_barrier()

    return sc_kernel(feat, idx_chunks)


def _tc_dense(feat, nsum, actions, w_top, w_bot, f1w, f1b, f2w, f2b, f3w, f3b):
    def body(actions_ref, feat_ref, nsum_ref, wt_ref, wb_ref,
             f1w_ref, f1b_ref, f2w_ref, f2b_ref, f3w_ref, f3b_ref,
             out_ref, xbuf):
        b = pl.program_id(0)
        c = pl.program_id(1)

        @pl.when(jnp.logical_and(b == 0, c == 0))
        def _init():
            xbuf[...] = jnp.zeros_like(xbuf)

        f = feat_ref[0]
        s = nsum_ref[0]
        e = jnp.dot(f, wt_ref[...], preferred_element_type=jnp.float32)
        e = e + jnp.dot(s, wb_ref[...], preferred_element_type=jnp.float32)
        e = jnp.maximum(e, 0.0)

        part_sum = jnp.sum(e, axis=0, keepdims=True)
        act = actions_ref[b]
        rows = lax.broadcasted_iota(jnp.int32, (R, OUT), 0) + c * R
        mask = (rows == act).astype(jnp.float32)
        part_act = jnp.sum(e * mask, axis=0, keepdims=True)
        upd = jnp.concatenate([part_sum, part_act], axis=1)
        xbuf[pl.ds(b, 1), :] = xbuf[pl.ds(b, 1), :] + upd

        @pl.when(jnp.logical_and(b == B - 1, c == NC - 1))
        def _head():
            scale = jnp.concatenate(
                [jnp.full((1, OUT), 1.0 / N, jnp.float32),
                 jnp.ones((1, OUT), jnp.float32)], axis=1)
            x = xbuf[...] * scale
            h = jnp.dot(x, f1w_ref[...], preferred_element_type=jnp.float32)
            h = jnp.maximum(h + f1b_ref[...], 0.0)
            h = jnp.dot(h, f2w_ref[...], preferred_element_type=jnp.float32)
            h = jnp.maximum(h + f2b_ref[...], 0.0)
            o = jnp.dot(h, f3w_ref[...], preferred_element_type=jnp.float32)
            out_ref[...] = o + f3b_ref[...]

    return pl.pallas_call(
        body,
        grid=(B, NC),
        in_specs=[
            pl.BlockSpec(memory_space=pltpu.SMEM),
            pl.BlockSpec((1, R, D), lambda b, c: (b, c, 0)),
            pl.BlockSpec((1, R, D), lambda b, c: (b, c, 0)),
            pl.BlockSpec((D, OUT), lambda b, c: (0, 0)),
            pl.BlockSpec((D, OUT), lambda b, c: (0, 0)),
            pl.BlockSpec((2 * OUT, HID), lambda b, c: (0, 0)),
            pl.BlockSpec((1, HID), lambda b, c: (0, 0)),
            pl.BlockSpec((HID, HID), lambda b, c: (0, 0)),
            pl.BlockSpec((1, HID), lambda b, c: (0, 0)),
            pl.BlockSpec((HID, OUT), lambda b, c: (0, 0)),
            pl.BlockSpec((1, OUT), lambda b, c: (0, 0)),
        ],
        out_specs=pl.BlockSpec((8, OUT), lambda b, c: (0, 0)),
        out_shape=jax.ShapeDtypeStruct((8, OUT), jnp.float32),
        scratch_shapes=[pltpu.VMEM((8, 2 * OUT), jnp.float32)],
    )(actions, feat, nsum, w_top, w_bot, f1w, f1b, f2w, f2b, f3w, f3b)


def kernel(actions, features, adj_lists, nodes, W_sage, fc1_w, fc1_b, fc2_w, fc2_b, fc3_w, fc3_b):
    del nodes
    adj = adj_lists.astype(jnp.int32)
    adj_pad = jnp.pad(adj, ((0, 0), (0, N_PAD - N), (0, 0)))
    idx_chunks = adj_pad.reshape(B, CH_PER_G, C, K).transpose(0, 1, 3, 2)

    nsum = _sc_neighbor_sum(features, idx_chunks)

    w_top = W_sage[:D]
    w_bot = W_sage[D:] * (1.0 / K)
    f3w = jnp.pad(fc3_w, ((0, 0), (0, OUT - 1)))
    f3b = jnp.pad(fc3_b, (0, OUT - 1)).reshape(1, OUT)

    out8 = _tc_dense(features, nsum, actions.astype(jnp.int32),
                     w_top, w_bot,
                     fc1_w, fc1_b.reshape(1, HID),
                     fc2_w, fc2_b.reshape(1, HID),
                     f3w, f3b)
    return out8[:B, :1]


# speedup vs baseline: 6.4686x; 1.0133x over previous
import functools

import jax
import jax.numpy as jnp
from jax import lax
from jax.experimental import pallas as pl
from jax.experimental.pallas import tpu as pltpu
from jax.experimental.pallas import tpu_sc as plsc

B, N, K, D = 4, 10000, 32, 128
OUT, HID = 128, 256
NSUB = 16
C = 128
N_PAD = 10240
CH_PER_G = N_PAD // C
CH_PER_W = CH_PER_G // NSUB
G_PER_CORE = B // 2
STG = 640
R = 1000
NC = N // R


def _sc_neighbor_sum(feat, idx_chunks):
    mesh = plsc.VectorSubcoreMesh(core_axis_name="c", subcore_axis_name="s")

    @functools.partial(
        pl.kernel,
        out_type=jax.ShapeDtypeStruct((B, N_PAD, D), jnp.float32),
        mesh=mesh,
        scratch_types=[
            pltpu.VMEM_SHARED((N, D), jnp.float32),
            pltpu.VMEM((2, K, C), jnp.int32),
            pltpu.VMEM((C, D), jnp.float32),
            pltpu.VMEM((C, D), jnp.float32),
            pltpu.SemaphoreType.DMA,
            pltpu.SemaphoreType.DMA,
        ],
    )
    def sc_kernel(feat_hbm, idx_hbm, out_hbm, table, idx_v, acc0, acc1, sem0, sem1):
        cid = lax.axis_index("c")
        sid = lax.axis_index("s")
        accs = (acc0, acc1)
        sems = (sem0, sem1)

        def zero(acc):
            z = jnp.zeros((16,), jnp.float32)

            def zrow(r, c2):
                for i in range(D // 16):
                    acc[r, pl.ds(i * 16, 16)] = z
                return c2

            lax.fori_loop(0, C, zrow, 0)

        def fire(p, acc, sem):
            def fk(k, c2):
                pltpu.async_copy(table.at[idx_v.at[p, k]], acc, sem, add=True)
                return c2

            lax.fori_loop(0, K, fk, 0)

        def drain(feat_hbm, acc, sem):
            def dk(k, c2):
                pltpu.make_async_copy(feat_hbm.at[0, pl.ds(0, C)], acc, sem).wait()
                return c2

            lax.fori_loop(0, K, dk, 0)

        for g in range(G_PER_CORE):
            b = cid * G_PER_CORE + g

            @pl.when(sid < NSUB - 1)
            def _stage_full():
                pltpu.sync_copy(feat_hbm.at[b, pl.ds(sid * STG, STG)],
                                table.at[pl.ds(sid * STG, STG)])

            @pl.when(sid == NSUB - 1)
            def _stage_tail():
                pltpu.sync_copy(feat_hbm.at[b, pl.ds((NSUB - 1) * STG, N - (NSUB - 1) * STG)],
                                table.at[pl.ds((NSUB - 1) * STG, N - (NSUB - 1) * STG)])

            pltpu.sync_copy(idx_hbm.at[b, sid * CH_PER_W], idx_v.at[0])
            pltpu.sync_copy(idx_hbm.at[b, sid * CH_PER_W + 1], idx_v.at[1])
            zero(acc0)
            zero(acc1)
            plsc.subcore_barrier()

            fire(0, accs[0], sems[0])
            fire(1, accs[1], sems[1])
            for j in range(2, CH_PER_W + 2):
                p = j % 2
                drain(feat_hbm, accs[p], sems[p])
                chunk = sid * CH_PER_W + (j - 2)
                pltpu.sync_copy(accs[p], out_hbm.at[b, pl.ds(chunk * C, C)])
                if j < CH_PER_W:
                    pltpu.sync_copy(idx_hbm.at[b, sid * CH_PER_W + j], idx_v.at[p])
                    zero(accs[p])
                    fire(p, accs[p], sems[p])

            plsc.subcore_barrier()

    return sc_kernel(feat, idx_chunks)


def _tc_dense(feat, nsum, actions, w_top, w_bot, f1w, f1b, f2w, f2b, f3w, f3b):
    def body(actions_ref, feat_ref, nsum_ref, wt_ref, wb_ref,
             f1w_ref, f1b_ref, f2w_ref, f2b_ref, f3w_ref, f3b_ref,
             out_ref, xbuf):
        b = pl.program_id(0)
        c = pl.program_id(1)

        @pl.when(jnp.logical_and(b == 0, c == 0))
        def _init():
            xbuf[...] = jnp.zeros_like(xbuf)

        f = feat_ref[0]
        s = nsum_ref[0]
        e = jnp.dot(f, wt_ref[...], preferred_element_type=jnp.float32)
        e = e + jnp.dot(s, wb_ref[...], preferred_element_type=jnp.float32)
        e = jnp.maximum(e, 0.0)

        part_sum = jnp.sum(e, axis=0, keepdims=True)
        act = actions_ref[b]
        rows = lax.broadcasted_iota(jnp.int32, (R, OUT), 0) + c * R
        mask = (rows == act).astype(jnp.float32)
        part_act = jnp.sum(e * mask, axis=0, keepdims=True)
        upd = jnp.concatenate([part_sum, part_act], axis=1)
        xbuf[pl.ds(b, 1), :] = xbuf[pl.ds(b, 1), :] + upd

        @pl.when(jnp.logical_and(b == B - 1, c == NC - 1))
        def _head():
            scale = jnp.concatenate(
                [jnp.full((1, OUT), 1.0 / N, jnp.float32),
                 jnp.ones((1, OUT), jnp.float32)], axis=1)
            x = xbuf[...] * scale
            h = jnp.dot(x, f1w_ref[...], preferred_element_type=jnp.float32)
            h = jnp.maximum(h + f1b_ref[...], 0.0)
            h = jnp.dot(h, f2w_ref[...], preferred_element_type=jnp.float32)
            h = jnp.maximum(h + f2b_ref[...], 0.0)
            o = jnp.dot(h, f3w_ref[...], preferred_element_type=jnp.float32)
            out_ref[...] = o + f3b_ref[...]

    return pl.pallas_call(
        body,
        grid=(B, NC),
        in_specs=[
            pl.BlockSpec(memory_space=pltpu.SMEM),
            pl.BlockSpec((1, R, D), lambda b, c: (b, c, 0)),
            pl.BlockSpec((1, R, D), lambda b, c: (b, c, 0)),
            pl.BlockSpec((D, OUT), lambda b, c: (0, 0)),
            pl.BlockSpec((D, OUT), lambda b, c: (0, 0)),
            pl.BlockSpec((2 * OUT, HID), lambda b, c: (0, 0)),
            pl.BlockSpec((1, HID), lambda b, c: (0, 0)),
            pl.BlockSpec((HID, HID), lambda b, c: (0, 0)),
            pl.BlockSpec((1, HID), lambda b, c: (0, 0)),
            pl.BlockSpec((HID, OUT), lambda b, c: (0, 0)),
            pl.BlockSpec((1, OUT), lambda b, c: (0, 0)),
        ],
        out_specs=pl.BlockSpec((8, OUT), lambda b, c: (0, 0)),
        out_shape=jax.ShapeDtypeStruct((8, OUT), jnp.float32),
        scratch_shapes=[pltpu.VMEM((8, 2 * OUT), jnp.float32)],
    )(actions, feat, nsum, w_top, w_bot, f1w, f1b, f2w, f2b, f3w, f3b)


def kernel(actions, features, adj_lists, nodes, W_sage, fc1_w, fc1_b, fc2_w, fc2_b, fc3_w, fc3_b):
    del nodes
    adj = adj_lists.astype(jnp.int32)
    adj_pad = jnp.pad(adj, ((0, 0), (0, N_PAD - N), (0, 0)))
    idx_chunks = adj_pad.reshape(B, CH_PER_G, C, K).transpose(0, 1, 3, 2)

    nsum = _sc_neighbor_sum(features, idx_chunks)

    w_top = W_sage[:D]
    w_bot = W_sage[D:] * (1.0 / K)
    f3w = jnp.pad(fc3_w, ((0, 0), (0, OUT - 1)))
    f3b = jnp.pad(fc3_b, (0, OUT - 1)).reshape(1, OUT)

    out8 = _tc_dense(features, nsum, actions.astype(jnp.int32),
                     w_top, w_bot,
                     fc1_w, fc1_b.reshape(1, HID),
                     fc2_w, fc2_b.reshape(1, HID),
                     f3w, f3b)
    return out8[:B, :1]


# speedup vs baseline: 6.7979x; 1.0509x over previous
import functools

import jax
import jax.numpy as jnp
from jax import lax
from jax.experimental import pallas as pl
from jax.experimental.pallas import tpu as pltpu
from jax.experimental.pallas import tpu_sc as plsc

B, N, K, D = 4, 10000, 32, 128
OUT, HID = 128, 256
NSUB = 16
C = 128
N_PAD = 10240
CH_PER_G = N_PAD // C
CH_PER_W = CH_PER_G // NSUB
STG = 640
R = 1000
NC = N // R


def _sc_neighbor_sum(feat, idx_chunks, h):
    mesh = plsc.VectorSubcoreMesh(core_axis_name="c", subcore_axis_name="s")

    @functools.partial(
        pl.kernel,
        out_type=jax.ShapeDtypeStruct((2, N_PAD, D), jnp.float32),
        mesh=mesh,
        scratch_types=[
            pltpu.VMEM_SHARED((N, D), jnp.float32),
            pltpu.VMEM((2, K, C), jnp.int32),
            pltpu.VMEM((C, D), jnp.float32),
            pltpu.VMEM((C, D), jnp.float32),
            pltpu.SemaphoreType.DMA,
            pltpu.SemaphoreType.DMA,
        ],
    )
    def sc_kernel(feat_hbm, idx_hbm, out_hbm, table, idx_v, acc0, acc1, sem0, sem1):
        cid = lax.axis_index("c")
        sid = lax.axis_index("s")
        b = 2 * cid + h
        accs = (acc0, acc1)
        sems = (sem0, sem1)

        def zero(acc):
            z = jnp.zeros((16,), jnp.float32)

            def zrow(r, c2):
                for i in range(D // 16):
                    acc[r, pl.ds(i * 16, 16)] = z
                return c2

            lax.fori_loop(0, C, zrow, 0)

        def fire(p, acc, sem):
            def fk(k, c2):
                pltpu.async_copy(table.at[idx_v.at[p, k]], acc, sem, add=True)
                return c2

            lax.fori_loop(0, K, fk, 0)

        def drain(acc, sem):
            def dk(k, c2):
                pltpu.make_async_copy(feat_hbm.at[0, pl.ds(0, C)], acc, sem).wait()
                return c2

            lax.fori_loop(0, K, dk, 0)

        @pl.when(sid < NSUB - 1)
        def _stage_full():
            pltpu.sync_copy(feat_hbm.at[b, pl.ds(sid * STG, STG)],
                            table.at[pl.ds(sid * STG, STG)])

        @pl.when(sid == NSUB - 1)
        def _stage_tail():
            pltpu.sync_copy(feat_hbm.at[b, pl.ds((NSUB - 1) * STG, N - (NSUB - 1) * STG)],
                            table.at[pl.ds((NSUB - 1) * STG, N - (NSUB - 1) * STG)])

        pltpu.sync_copy(idx_hbm.at[b, sid * CH_PER_W], idx_v.at[0])
        pltpu.sync_copy(idx_hbm.at[b, sid * CH_PER_W + 1], idx_v.at[1])
        zero(acc0)
        zero(acc1)
        plsc.subcore_barrier()

        fire(0, accs[0], sems[0])
        fire(1, accs[1], sems[1])
        for j in range(2, CH_PER_W + 2):
            p = j % 2
            drain(accs[p], sems[p])
            chunk = sid * CH_PER_W + (j - 2)
            pltpu.sync_copy(accs[p], out_hbm.at[cid, pl.ds(chunk * C, C)])
            if j < CH_PER_W:
                pltpu.sync_copy(idx_hbm.at[b, sid * CH_PER_W + j], idx_v.at[p])
                zero(accs[p])
                fire(p, accs[p], sems[p])

    return sc_kernel(feat, idx_chunks)


def _tc_dense(feat, nsum, actions, w_top, w_bot, h, xprev,
              f1w, f1b, f2w, f2b, f3w, f3b):
    final = h == 1

    def body(actions_ref, feat_ref, nsum_ref, wt_ref, wb_ref, xprev_ref,
             f1w_ref, f1b_ref, f2w_ref, f2b_ref, f3w_ref, f3b_ref,
             out_ref, xbuf):
        i = pl.program_id(0)
        c = pl.program_id(1)

        @pl.when(jnp.logical_and(i == 0, c == 0))
        def _init():
            xbuf[...] = xprev_ref[...]

        f = feat_ref[0]
        s = nsum_ref[0]
        e = jnp.dot(f, wt_ref[...], preferred_element_type=jnp.float32)
        e = e + jnp.dot(s, wb_ref[...], preferred_element_type=jnp.float32)
        e = jnp.maximum(e, 0.0)

        part_sum = jnp.sum(e, axis=0, keepdims=True)
        act = actions_ref[2 * i + h]
        rows = lax.broadcasted_iota(jnp.int32, (R, OUT), 0) + c * R
        mask = (rows == act).astype(jnp.float32)
        part_act = jnp.sum(e * mask, axis=0, keepdims=True)
        upd = jnp.concatenate([part_sum, part_act], axis=1)
        row = 2 * i + h
        xbuf[pl.ds(row, 1), :] = xbuf[pl.ds(row, 1), :] + upd

        @pl.when(jnp.logical_and(i == 1, c == NC - 1))
        def _tail():
            if final:
                scale = jnp.concatenate(
                    [jnp.full((1, OUT), 1.0 / N, jnp.float32),
                     jnp.ones((1, OUT), jnp.float32)], axis=1)
                x = xbuf[...] * scale
                hh = jnp.dot(x, f1w_ref[...], preferred_element_type=jnp.float32)
                hh = jnp.maximum(hh + f1b_ref[...], 0.0)
                hh = jnp.dot(hh, f2w_ref[...], preferred_element_type=jnp.float32)
                hh = jnp.maximum(hh + f2b_ref[...], 0.0)
                o = jnp.dot(hh, f3w_ref[...], preferred_element_type=jnp.float32)
                out_ref[...] = o + f3b_ref[...]
            else:
                out_ref[...] = xbuf[...]

    out_cols = OUT if final else 2 * OUT
    return pl.pallas_call(
        body,
        grid=(2, NC),
        in_specs=[
            pl.BlockSpec(memory_space=pltpu.SMEM),
            pl.BlockSpec((1, R, D), lambda i, c: (2 * i + h, c, 0)),
            pl.BlockSpec((1, R, D), lambda i, c: (i, c, 0)),
            pl.BlockSpec((D, OUT), lambda i, c: (0, 0)),
            pl.BlockSpec((D, OUT), lambda i, c: (0, 0)),
            pl.BlockSpec((8, 2 * OUT), lambda i, c: (0, 0)),
            pl.BlockSpec((2 * OUT, HID), lambda i, c: (0, 0)),
            pl.BlockSpec((1, HID), lambda i, c: (0, 0)),
            pl.BlockSpec((HID, HID), lambda i, c: (0, 0)),
            pl.BlockSpec((1, HID), lambda i, c: (0, 0)),
            pl.BlockSpec((HID, OUT), lambda i, c: (0, 0)),
            pl.BlockSpec((1, OUT), lambda i, c: (0, 0)),
        ],
        out_specs=pl.BlockSpec((8, out_cols), lambda i, c: (0, 0)),
        out_shape=jax.ShapeDtypeStruct((8, out_cols), jnp.float32),
        scratch_shapes=[pltpu.VMEM((8, 2 * OUT), jnp.float32)],
    )(actions, feat, nsum, w_top, w_bot, xprev,
      f1w, f1b, f2w, f2b, f3w, f3b)


def kernel(actions, features, adj_lists, nodes, W_sage, fc1_w, fc1_b, fc2_w, fc2_b, fc3_w, fc3_b):
    del nodes
    adj = adj_lists.astype(jnp.int32)
    adj_pad = jnp.pad(adj, ((0, 0), (0, N_PAD - N), (0, 0)))
    idx_chunks = adj_pad.reshape(B, CH_PER_G, C, K).transpose(0, 1, 3, 2)

    w_top = W_sage[:D]
    w_bot = W_sage[D:] * (1.0 / K)
    f3w = jnp.pad(fc3_w, ((0, 0), (0, OUT - 1)))
    f3b = jnp.pad(fc3_b, (0, OUT - 1)).reshape(1, OUT)
    acts = actions.astype(jnp.int32)
    f1b2 = fc1_b.reshape(1, HID)
    f2b2 = fc2_b.reshape(1, HID)

    nsum0 = _sc_neighbor_sum(features, idx_chunks, 0)
    nsum1 = _sc_neighbor_sum(features, idx_chunks, 1)

    xzero = jnp.zeros((8, 2 * OUT), jnp.float32)
    xpart = _tc_dense(features, nsum0, acts, w_top, w_bot, 0, xzero,
                      fc1_w, f1b2, fc2_w, f2b2, f3w, f3b)
    out8 = _tc_dense(features, nsum1, acts, w_top, w_bot, 1, xpart,
                     fc1_w, f1b2, fc2_w, f2b2, f3w, f3b)
    return out8[:B, :1]
